# Initial kernel scaffold; baseline (speedup 1.0000x reference)
import jax
import jax.numpy as jnp
from jax.experimental import pallas as pl


def kernel(x, edge_index, batch, W0, as0, ad0, b0, g0, be0, W1, as1, ad1, b1, g1, be1, W2, as2, ad2, b2, g2, be2, W3, as3, ad3, b3, g3, be3, skip_W, skip_b, c1W, c1b, c2W, c2b, c3W, c3b):
    raise NotImplementedError("write your pallas kernel here")


# speedup vs baseline: 19.0053x; 19.0053x over previous
import functools

import jax
import jax.numpy as jnp
from jax import lax
from jax.experimental import pallas as pl
from jax.experimental.pallas import tpu as pltpu
from jax.experimental.pallas import tpu_sc as plsc

N = 10000
E = 320000
HID = 128
G = 16
EXT = 144

NC = 2
NS = 16
NW = NC * NS
EPT = E // NW
K = 80
NCHUNK = EPT // K
EROWS = E // K

_mesh = plsc.VectorSubcoreMesh(
    core_axis_name="c", subcore_axis_name="s", num_cores=NC, num_subcores=NS)


def _edge_body(hext, src2, dst2, asrc, adst, zeros,
               msg_out, ex_out,
               asrc_v, adst_v, src_c, dst_c, ex_c, rows_v, msg_s, sem):
    c = lax.axis_index("c")
    s = lax.axis_index("s")
    wid = c * NS + s
    base = wid * NCHUNK

    pltpu.sync_copy(asrc, asrc_v)
    pltpu.sync_copy(adst, adst_v)
    rows_per_tile = N // NS
    pltpu.sync_copy(zeros.at[pl.ds(s * rows_per_tile, rows_per_tile)],
                    msg_s.at[pl.ds(s * rows_per_tile, rows_per_tile)])
    plsc.subcore_barrier()

    def chunk(i, carry):
        pltpu.sync_copy(src2.at[base + i], src_c)
        pltpu.sync_copy(dst2.at[base + i], dst_c)
        pltpu.async_copy(hext.at[src_c], rows_v, sem).wait()
        for k in range(K // 16):
            s16 = src_c[pl.ds(k * 16, 16)]
            d16 = dst_c[pl.ds(k * 16, 16)]
            al = (plsc.load_gather(asrc_v, [s16])
                  + plsc.load_gather(adst_v, [d16]))
            al = jnp.where(al > 0.0, al, al * jnp.float32(0.2))
            ex_c[pl.ds(k * 16, 16)] = jnp.exp(al)

        def scale(j, carry2):
            exj = plsc.load_gather(ex_c, [jnp.full((16,), j, jnp.int32)])
            for col in range(EXT // 16):
                rows_v[j, pl.ds(col * 16, 16)] = (
                    rows_v[j, pl.ds(col * 16, 16)] * exj)
            return carry2
        lax.fori_loop(0, K, scale, 0)

        pltpu.sync_copy(rows_v, msg_s.at[dst_c], add=True)
        pltpu.sync_copy(ex_c, ex_out.at[base + i])
        return carry
    lax.fori_loop(0, NCHUNK, chunk, 0)

    plsc.subcore_barrier()
    pltpu.sync_copy(msg_s.at[pl.ds(s * rows_per_tile, rows_per_tile)],
                    msg_out.at[c, pl.ds(s * rows_per_tile, rows_per_tile)])


def _edge_pass(hext, src2, dst2, asrc, adst, zeros):
    f = pl.kernel(
        _edge_body,
        out_type=(jax.ShapeDtypeStruct((NC, N, EXT), jnp.float32),
                  jax.ShapeDtypeStruct((EROWS, K), jnp.float32)),
        mesh=_mesh,
        scratch_types=[
            pltpu.VMEM((N,), jnp.float32),
            pltpu.VMEM((N,), jnp.float32),
            pltpu.VMEM((K,), jnp.int32),
            pltpu.VMEM((K,), jnp.int32),
            pltpu.VMEM((K,), jnp.float32),
            pltpu.VMEM((K, EXT), jnp.float32),
            pltpu.VMEM_SHARED((N, EXT), jnp.float32),
            pltpu.SemaphoreType.DMA,
        ],
        compiler_params=pltpu.CompilerParams(use_tc_tiling_on_sc=False, needs_layout_passes=False),
    )
    return f(hext, src2, dst2, asrc, adst, zeros)


def _att_body(ex2, dst2, inv, att_out, inv_v, exc, dstc, attc):
    c = lax.axis_index("c")
    s = lax.axis_index("s")
    base = (c * NS + s) * NCHUNK

    pltpu.sync_copy(inv, inv_v)
    pltpu.sync_copy(ex2.at[pl.ds(base, NCHUNK)], exc)
    pltpu.sync_copy(dst2.at[pl.ds(base, NCHUNK)], dstc)

    def chunk(i, carry):
        for k in range(K // 16):
            d16 = dstc[i, pl.ds(k * 16, 16)]
            iv = plsc.load_gather(inv_v, [d16])
            attc[i, pl.ds(k * 16, 16)] = exc[i, pl.ds(k * 16, 16)] * iv
        return carry
    lax.fori_loop(0, NCHUNK, chunk, 0)
    pltpu.sync_copy(attc, att_out.at[pl.ds(base, NCHUNK)])


def _att_pass(ex2, dst2, inv):
    f = pl.kernel(
        _att_body,
        out_type=jax.ShapeDtypeStruct((EROWS, K), jnp.float32),
        mesh=_mesh,
        scratch_types=[
            pltpu.VMEM((N,), jnp.float32),
            pltpu.VMEM((NCHUNK, K), jnp.float32),
            pltpu.VMEM((NCHUNK, K), jnp.int32),
            pltpu.VMEM((NCHUNK, K), jnp.float32),
        ],
        compiler_params=pltpu.CompilerParams(use_tc_tiling_on_sc=False, needs_layout_passes=False),
    )
    return f(ex2, dst2, inv)


_BLK = 1000
_NBLK = N // _BLK


def _pre_body(hin, W, a_s, a_d, hext, asrc, adst):
    h = jnp.dot(hin[...], W[...], preferred_element_type=jnp.float32)
    asrc[...] = jnp.sum(h * a_s[...][None, :], axis=1, keepdims=True)
    adst[...] = jnp.sum(h * a_d[...][None, :], axis=1, keepdims=True)
    hext[...] = jnp.concatenate(
        [h, jnp.ones((h.shape[0], 1), jnp.float32),
         jnp.zeros((h.shape[0], EXT - HID - 1), jnp.float32)], axis=1)


def _pre(hin, W, a_s, a_d):
    fin = hin.shape[1]
    return pl.pallas_call(
        _pre_body,
        grid=(_NBLK,),
        in_specs=[
            pl.BlockSpec((_BLK, fin), lambda i: (i, 0)),
            pl.BlockSpec((fin, HID), lambda i: (0, 0)),
            pl.BlockSpec((HID,), lambda i: (0,)),
            pl.BlockSpec((HID,), lambda i: (0,)),
        ],
        out_specs=[
            pl.BlockSpec((_BLK, EXT), lambda i: (i, 0)),
            pl.BlockSpec((_BLK, 1), lambda i: (i, 0)),
            pl.BlockSpec((_BLK, 1), lambda i: (i, 0)),
        ],
        out_shape=[
            jax.ShapeDtypeStruct((N, EXT), jnp.float32),
            jax.ShapeDtypeStruct((N, 1), jnp.float32),
            jax.ShapeDtypeStruct((N, 1), jnp.float32),
        ],
    )(hin, W, a_s, a_d)


def _skip_body(x, W, b, out):
    out[...] = (jnp.dot(x[...], W[...], preferred_element_type=jnp.float32)
                + b[...][None, :])


def _skip(x, W, b):
    return pl.pallas_call(
        _skip_body,
        grid=(_NBLK,),
        in_specs=[
            pl.BlockSpec((_BLK, x.shape[1]), lambda i: (i, 0)),
            pl.BlockSpec((x.shape[1], HID), lambda i: (0, 0)),
            pl.BlockSpec((HID,), lambda i: (0,)),
        ],
        out_specs=pl.BlockSpec((_BLK, HID), lambda i: (i, 0)),
        out_shape=jax.ShapeDtypeStruct((N, HID), jnp.float32),
    )(x, W, b)


def _post_body(msg, hext, asrc, adst, res, b, g, be, hnext, inv, attl,
               *, last):
    m = msg[0] + msg[1]
    hx = hext[...]
    h = hx[:, :HID]
    al = asrc[...][:, 0] + adst[...][:, 0]
    al = jnp.where(al > 0.0, al, al * 0.2)
    exl = jnp.exp(al)
    den = m[:, HID] + exl
    iv = 1.0 / (den + 1e-16)
    gat = (m[:, :HID] + exl[:, None] * h) * iv[:, None] + b[...][None, :]
    mu = jnp.mean(gat, axis=1, keepdims=True)
    var = jnp.mean((gat - mu) ** 2, axis=1, keepdims=True)
    y = (gat - mu) / jnp.sqrt(var + 1e-5) * g[...][None, :] + be[...][None, :]
    if not last:
        y = jnp.where(y > 0.0, y, jnp.exp(y) - 1.0)
    hnext[...] = y + res[...]
    inv[...] = iv[:, None]
    attl[...] = (exl * iv)[:, None]


def _post(msg, hext, asrc, adst, res, b, g, be, last):
    return pl.pallas_call(
        functools.partial(_post_body, last=last),
        grid=(_NBLK,),
        in_specs=[
            pl.BlockSpec((NC, _BLK, EXT), lambda i: (0, i, 0)),
            pl.BlockSpec((_BLK, EXT), lambda i: (i, 0)),
            pl.BlockSpec((_BLK, 1), lambda i: (i, 0)),
            pl.BlockSpec((_BLK, 1), lambda i: (i, 0)),
            pl.BlockSpec((_BLK, HID), lambda i: (i, 0)),
            pl.BlockSpec((HID,), lambda i: (0,)),
            pl.BlockSpec((HID,), lambda i: (0,)),
            pl.BlockSpec((HID,), lambda i: (0,)),
        ],
        out_specs=[
            pl.BlockSpec((_BLK, HID), lambda i: (i, 0)),
            pl.BlockSpec((_BLK, 1), lambda i: (i, 0)),
            pl.BlockSpec((_BLK, 1), lambda i: (i, 0)),
        ],
        out_shape=[
            jax.ShapeDtypeStruct((N, HID), jnp.float32),
            jax.ShapeDtypeStruct((N, 1), jnp.float32),
            jax.ShapeDtypeStruct((N, 1), jnp.float32),
        ],
    )(msg, hext, asrc, adst, res, b, g, be)


def _pool_body(h, batch, c1W, c1b, c2W, c2b, c3W, c3b, out,
               sums, maxs, cnt):
    step = pl.program_id(0)

    @pl.when(step == 0)
    def _init():
        sums[...] = jnp.zeros((G, HID), jnp.float32)
        cnt[...] = jnp.zeros((G, HID), jnp.float32)
        maxs[...] = jnp.full((G, HID), -jnp.inf, jnp.float32)

    hb = h[...]
    bb = batch[...][:, 0]
    onehot = (bb[:, None]
              == lax.broadcasted_iota(jnp.int32, (1, G), 1)).astype(jnp.float32)
    sums[...] += lax.dot_general(onehot, hb, (((0,), (0,)), ((), ())),
                                 preferred_element_type=jnp.float32)
    cnt[...] += jnp.broadcast_to(jnp.sum(onehot, axis=0)[:, None], (G, HID))
    for gi in range(G):
        mg = jnp.max(jnp.where((bb == gi)[:, None], hb, -jnp.inf),
                     axis=0, keepdims=True)
        maxs[pl.ds(gi, 1), :] = jnp.maximum(maxs[pl.ds(gi, 1), :], mg)

    @pl.when(step == pl.num_programs(0) - 1)
    def _fin():
        xmean = sums[...] / jnp.maximum(cnt[...], 1.0)
        xmax = maxs[...]
        xmax = jnp.where(jnp.isfinite(xmax), xmax, 0.0)
        z = jnp.concatenate([xmean, xmax], axis=1)
        z = jnp.maximum(
            jnp.dot(z, c1W[...], preferred_element_type=jnp.float32)
            + c1b[...][None, :], 0.0)
        z = jnp.maximum(
            jnp.dot(z, c2W[...], preferred_element_type=jnp.float32)
            + c2b[...][None, :], 0.0)
        out[...] = (jnp.dot(z, c3W[...], preferred_element_type=jnp.float32)
                    + c3b[...][None, :])


def _pool_mlp(h, batch, c1W, c1b, c2W, c2b, c3W, c3b):
    return pl.pallas_call(
        _pool_body,
        grid=(_NBLK,),
        in_specs=[
            pl.BlockSpec((_BLK, HID), lambda i: (i, 0)),
            pl.BlockSpec((_BLK, 1), lambda i: (i, 0)),
            pl.BlockSpec((2 * HID, HID), lambda i: (0, 0)),
            pl.BlockSpec((HID,), lambda i: (0,)),
            pl.BlockSpec((HID, HID // 2), lambda i: (0, 0)),
            pl.BlockSpec((HID // 2,), lambda i: (0,)),
            pl.BlockSpec((HID // 2, 10), lambda i: (0, 0)),
            pl.BlockSpec((10,), lambda i: (0,)),
        ],
        out_specs=pl.BlockSpec((G, 10), lambda i: (0, 0)),
        out_shape=jax.ShapeDtypeStruct((G, 10), jnp.float32),
        scratch_shapes=[
            pltpu.VMEM((G, HID), jnp.float32),
            pltpu.VMEM((G, HID), jnp.float32),
            pltpu.VMEM((G, HID), jnp.float32),
        ],
    )(h, batch, c1W, c1b, c2W, c2b, c3W, c3b)


def kernel(x, edge_index, batch,
           W0, as0, ad0, b0, g0, be0,
           W1, as1, ad1, b1, g1, be1,
           W2, as2, ad2, b2, g2, be2,
           W3, as3, ad3, b3, g3, be3,
           skip_W, skip_b, c1W, c1b, c2W, c2b, c3W, c3b):
    src2 = edge_index[0].reshape(EROWS, K)
    dst2 = edge_index[1].reshape(EROWS, K)
    zeros = jnp.zeros((N, EXT), jnp.float32)

    Ws = [W0, W1, W2, W3]
    ass = [as0, as1, as2, as3]
    ads = [ad0, ad1, ad2, ad3]
    bs = [b0, b1, b2, b3]
    gs = [g0, g1, g2, g3]
    bes = [be0, be1, be2, be3]

    skip = _skip(x, skip_W, skip_b)
    h = x
    atts = []
    for i in range(4):
        hext, asrc, adst = _pre(h, Ws[i], ass[i], ads[i])
        msg, ex2 = _edge_pass(hext, src2, dst2, asrc.reshape(N),
                              adst.reshape(N), zeros)
        res = skip if i == 0 else h
        hnext, inv, attl = _post(msg, hext, asrc, adst, res,
                                 bs[i], gs[i], bes[i], last=(i == 3))
        att_e = _att_pass(ex2, dst2, inv.reshape(N))
        atts.append(jnp.concatenate([att_e.reshape(E),
                                     attl.reshape(N)])[:, None])
        h = hnext

    out = _pool_mlp(h, batch.reshape(N, 1), c1W, c1b, c2W, c2b, c3W, c3b)
    return (out,) + tuple(atts)


# speedup vs baseline: 34.9195x; 1.8374x over previous
import functools

import jax
import jax.numpy as jnp
from jax import lax
from jax.experimental import pallas as pl
from jax.experimental.pallas import tpu as pltpu
from jax.experimental.pallas import tpu_sc as plsc

N = 10000
E = 320000
HID = 128
G = 16
EXT = 144

NC = 2
NS = 16
NW = NC * NS
EPT = E // NW
K = 80
NCHUNK = EPT // K
EROWS = E // K

_mesh = plsc.VectorSubcoreMesh(
    core_axis_name="c", subcore_axis_name="s", num_cores=NC, num_subcores=NS)


def _edge_body(hext, packed, adst, zeros,
               msg_out, ex_out,
               packed_v, src_c, dst_c, ex_c, adg_c, rows_v, msg_s,
               sem_g, sem_s, sem_e):
    c = lax.axis_index("c")
    s = lax.axis_index("s")
    wid = c * NS + s
    base = wid * NCHUNK
    iota = lax.iota(jnp.int32, 16)

    pltpu.sync_copy(packed.at[pl.ds(wid * EPT, EPT)], packed_v)
    rows_per_tile = N // NS
    pltpu.sync_copy(zeros.at[pl.ds(s * rows_per_tile, rows_per_tile)],
                    msg_s.at[pl.ds(s * rows_per_tile, rows_per_tile)])
    plsc.subcore_barrier()

    def unpack(i, b):
        for k in range(K // 16):
            p16 = packed_v[pl.ds(i * K + k * 16, 16)]
            src_c[b, pl.ds(k * 16, 16)] = p16 & jnp.int32(0xFFFF)
            dst_c[b, pl.ds(k * 16, 16)] = lax.shift_right_logical(
                p16, jnp.int32(16))

    def issue_gather(i, b):
        pltpu.async_copy(hext.at[src_c.at[b]], rows_v.at[b], sem_g.at[b])
        pltpu.async_copy(adst.at[dst_c.at[b]], adg_c.at[b], sem_g.at[b])

    def wait_gather(b):
        pltpu.make_async_copy(hext.at[src_c.at[b]], rows_v.at[b],
                              sem_g.at[b]).wait()
        pltpu.make_async_copy(adst.at[dst_c.at[b]], adg_c.at[b],
                              sem_g.at[b]).wait()

    def wait_scatter(b):
        pltpu.make_async_copy(rows_v.at[b], msg_s.at[dst_c.at[b]],
                              sem_s.at[b]).wait()

    def wait_ex(i, b):
        pltpu.make_async_copy(ex_c.at[b], ex_out.at[base + i],
                              sem_e.at[b]).wait()

    def step(i, cur, nxt, prev, prev2, nxt_ok):
        if prev:
            wait_scatter(nxt)
        if nxt_ok:
            unpack(i + 1, nxt)
            issue_gather(i + 1, nxt)
        wait_gather(cur)
        if prev2:
            wait_ex(i - 2, cur)
        for k in range(K // 16):
            a1 = plsc.load_gather(
                rows_v.at[cur], [k * 16 + iota, jnp.full((16,), 129,
                                                         jnp.int32)])
            al = a1 + adg_c[cur, pl.ds(k * 16, 16)]
            al = jnp.where(al > 0.0, al, al * jnp.float32(0.2))
            ex_c[cur, pl.ds(k * 16, 16)] = jnp.exp(al)
        pltpu.async_copy(ex_c.at[cur], ex_out.at[base + i], sem_e.at[cur])

        def scale(j, carry2):
            for u in range(4):
                e = j * 4 + u
                exj = plsc.load_gather(
                    ex_c.at[cur], [jnp.full((16,), e, jnp.int32)])
                for col in range(EXT // 16):
                    rows_v[cur, e, pl.ds(col * 16, 16)] = (
                        rows_v[cur, e, pl.ds(col * 16, 16)] * exj)
            return carry2
        lax.fori_loop(0, K // 4, scale, 0)
        pltpu.async_copy(rows_v.at[cur], msg_s.at[dst_c.at[cur]],
                         sem_s.at[cur], add=True)

    unpack(0, 0)
    issue_gather(0, 0)
    step(0, 0, 1, prev=False, prev2=False, nxt_ok=True)

    def pair(g, carry):
        step(2 * g + 1, 1, 0, prev=True, prev2=True, nxt_ok=True)
        step(2 * g + 2, 0, 1, prev=True, prev2=True, nxt_ok=True)
        return carry
    step(1, 1, 0, prev=True, prev2=False, nxt_ok=True)
    step(2, 0, 1, prev=True, prev2=True, nxt_ok=True)
    lax.fori_loop(1, (NCHUNK - 3) // 2, pair, 0)
    step(NCHUNK - 2, 1, 0, prev=True, prev2=True, nxt_ok=True)
    step(NCHUNK - 1, 0, 1, prev=True, prev2=True, nxt_ok=False)

    wait_scatter(0)
    wait_ex(NCHUNK - 2, 1)
    wait_ex(NCHUNK - 1, 0)

    plsc.subcore_barrier()
    pltpu.sync_copy(msg_s.at[pl.ds(s * rows_per_tile, rows_per_tile)],
                    msg_out.at[c, pl.ds(s * rows_per_tile, rows_per_tile)])


def _edge_pass(hext, packed, adst, zeros):
    f = pl.kernel(
        _edge_body,
        out_type=(jax.ShapeDtypeStruct((NC, N, EXT), jnp.float32),
                  jax.ShapeDtypeStruct((EROWS, K), jnp.float32)),
        mesh=_mesh,
        scratch_types=[
            pltpu.VMEM((EPT,), jnp.int32),
            pltpu.VMEM((2, K), jnp.int32),
            pltpu.VMEM((2, K), jnp.int32),
            pltpu.VMEM((2, K), jnp.float32),
            pltpu.VMEM((2, K), jnp.float32),
            pltpu.VMEM((2, K, EXT), jnp.float32),
            pltpu.VMEM_SHARED((N, EXT), jnp.float32),
            pltpu.SemaphoreType.DMA((2,)),
            pltpu.SemaphoreType.DMA((2,)),
            pltpu.SemaphoreType.DMA((2,)),
        ],
        compiler_params=pltpu.CompilerParams(use_tc_tiling_on_sc=False, needs_layout_passes=False),
    )
    return f(hext, packed, adst, zeros)


def _att_body(ex2, dst2, inv, att_out, inv_v, exc, dstc, attc):
    c = lax.axis_index("c")
    s = lax.axis_index("s")
    base = (c * NS + s) * NCHUNK

    pltpu.sync_copy(inv, inv_v)
    pltpu.sync_copy(ex2.at[pl.ds(base, NCHUNK)], exc)
    pltpu.sync_copy(dst2.at[pl.ds(base, NCHUNK)], dstc)

    def chunk(i, carry):
        for k in range(K // 16):
            d16 = dstc[i, pl.ds(k * 16, 16)]
            iv = plsc.load_gather(inv_v, [d16])
            attc[i, pl.ds(k * 16, 16)] = exc[i, pl.ds(k * 16, 16)] * iv
        return carry
    lax.fori_loop(0, NCHUNK, chunk, 0)
    pltpu.sync_copy(attc, att_out.at[pl.ds(base, NCHUNK)])


def _att_pass(ex2, dst2, inv):
    f = pl.kernel(
        _att_body,
        out_type=jax.ShapeDtypeStruct((EROWS, K), jnp.float32),
        mesh=_mesh,
        scratch_types=[
            pltpu.VMEM((N,), jnp.float32),
            pltpu.VMEM((NCHUNK, K), jnp.float32),
            pltpu.VMEM((NCHUNK, K), jnp.int32),
            pltpu.VMEM((NCHUNK, K), jnp.float32),
        ],
        compiler_params=pltpu.CompilerParams(use_tc_tiling_on_sc=False, needs_layout_passes=False),
    )
    return f(ex2, dst2, inv)


_BLK = 1000
_NBLK = N // _BLK


def _pre_body(hin, W, a_s, a_d, hext, asrc, adst):
    h = jnp.dot(hin[...], W[...], preferred_element_type=jnp.float32)
    av = jnp.sum(h * a_s[...][None, :], axis=1, keepdims=True)
    asrc[...] = av
    adst[...] = jnp.sum(h * a_d[...][None, :], axis=1, keepdims=True)
    hext[...] = jnp.concatenate(
        [h, jnp.ones((h.shape[0], 1), jnp.float32), av,
         jnp.zeros((h.shape[0], EXT - HID - 2), jnp.float32)], axis=1)


def _pre(hin, W, a_s, a_d):
    fin = hin.shape[1]
    return pl.pallas_call(
        _pre_body,
        grid=(_NBLK,),
        in_specs=[
            pl.BlockSpec((_BLK, fin), lambda i: (i, 0)),
            pl.BlockSpec((fin, HID), lambda i: (0, 0)),
            pl.BlockSpec((HID,), lambda i: (0,)),
            pl.BlockSpec((HID,), lambda i: (0,)),
        ],
        out_specs=[
            pl.BlockSpec((_BLK, EXT), lambda i: (i, 0)),
            pl.BlockSpec((_BLK, 1), lambda i: (i, 0)),
            pl.BlockSpec((_BLK, 1), lambda i: (i, 0)),
        ],
        out_shape=[
            jax.ShapeDtypeStruct((N, EXT), jnp.float32),
            jax.ShapeDtypeStruct((N, 1), jnp.float32),
            jax.ShapeDtypeStruct((N, 1), jnp.float32),
        ],
    )(hin, W, a_s, a_d)


def _skip_body(x, W, b, out):
    out[...] = (jnp.dot(x[...], W[...], preferred_element_type=jnp.float32)
                + b[...][None, :])


def _skip(x, W, b):
    return pl.pallas_call(
        _skip_body,
        grid=(_NBLK,),
        in_specs=[
            pl.BlockSpec((_BLK, x.shape[1]), lambda i: (i, 0)),
            pl.BlockSpec((x.shape[1], HID), lambda i: (0, 0)),
            pl.BlockSpec((HID,), lambda i: (0,)),
        ],
        out_specs=pl.BlockSpec((_BLK, HID), lambda i: (i, 0)),
        out_shape=jax.ShapeDtypeStruct((N, HID), jnp.float32),
    )(x, W, b)


def _post_body(msg, hext, asrc, adst, res, b, g, be, hnext, inv, attl,
               *, last):
    m = msg[0] + msg[1]
    hx = hext[...]
    h = hx[:, :HID]
    al = asrc[...][:, 0] + adst[...][:, 0]
    al = jnp.where(al > 0.0, al, al * 0.2)
    exl = jnp.exp(al)
    den = m[:, HID] + exl
    iv = 1.0 / (den + 1e-16)
    gat = (m[:, :HID] + exl[:, None] * h) * iv[:, None] + b[...][None, :]
    mu = jnp.mean(gat, axis=1, keepdims=True)
    var = jnp.mean((gat - mu) ** 2, axis=1, keepdims=True)
    y = (gat - mu) / jnp.sqrt(var + 1e-5) * g[...][None, :] + be[...][None, :]
    if not last:
        y = jnp.where(y > 0.0, y, jnp.exp(y) - 1.0)
    hnext[...] = y + res[...]
    inv[...] = iv[:, None]
    attl[...] = (exl * iv)[:, None]


def _post(msg, hext, asrc, adst, res, b, g, be, last):
    return pl.pallas_call(
        functools.partial(_post_body, last=last),
        grid=(_NBLK,),
        in_specs=[
            pl.BlockSpec((NC, _BLK, EXT), lambda i: (0, i, 0)),
            pl.BlockSpec((_BLK, EXT), lambda i: (i, 0)),
            pl.BlockSpec((_BLK, 1), lambda i: (i, 0)),
            pl.BlockSpec((_BLK, 1), lambda i: (i, 0)),
            pl.BlockSpec((_BLK, HID), lambda i: (i, 0)),
            pl.BlockSpec((HID,), lambda i: (0,)),
            pl.BlockSpec((HID,), lambda i: (0,)),
            pl.BlockSpec((HID,), lambda i: (0,)),
        ],
        out_specs=[
            pl.BlockSpec((_BLK, HID), lambda i: (i, 0)),
            pl.BlockSpec((_BLK, 1), lambda i: (i, 0)),
            pl.BlockSpec((_BLK, 1), lambda i: (i, 0)),
        ],
        out_shape=[
            jax.ShapeDtypeStruct((N, HID), jnp.float32),
            jax.ShapeDtypeStruct((N, 1), jnp.float32),
            jax.ShapeDtypeStruct((N, 1), jnp.float32),
        ],
    )(msg, hext, asrc, adst, res, b, g, be)


def _pool_body(h, batch, c1W, c1b, c2W, c2b, c3W, c3b, out,
               sums, maxs, cnt):
    step = pl.program_id(0)

    @pl.when(step == 0)
    def _init():
        sums[...] = jnp.zeros((G, HID), jnp.float32)
        cnt[...] = jnp.zeros((G, HID), jnp.float32)
        maxs[...] = jnp.full((G, HID), -jnp.inf, jnp.float32)

    hb = h[...]
    bb = batch[...][:, 0]
    onehot = (bb[:, None]
              == lax.broadcasted_iota(jnp.int32, (1, G), 1)).astype(jnp.float32)
    sums[...] += lax.dot_general(onehot, hb, (((0,), (0,)), ((), ())),
                                 preferred_element_type=jnp.float32)
    cnt[...] += jnp.broadcast_to(jnp.sum(onehot, axis=0)[:, None], (G, HID))
    for gi in range(G):
        mg = jnp.max(jnp.where((bb == gi)[:, None], hb, -jnp.inf),
                     axis=0, keepdims=True)
        maxs[pl.ds(gi, 1), :] = jnp.maximum(maxs[pl.ds(gi, 1), :], mg)

    @pl.when(step == pl.num_programs(0) - 1)
    def _fin():
        xmean = sums[...] / jnp.maximum(cnt[...], 1.0)
        xmax = maxs[...]
        xmax = jnp.where(jnp.isfinite(xmax), xmax, 0.0)
        z = jnp.concatenate([xmean, xmax], axis=1)
        z = jnp.maximum(
            jnp.dot(z, c1W[...], preferred_element_type=jnp.float32)
            + c1b[...][None, :], 0.0)
        z = jnp.maximum(
            jnp.dot(z, c2W[...], preferred_element_type=jnp.float32)
            + c2b[...][None, :], 0.0)
        out[...] = (jnp.dot(z, c3W[...], preferred_element_type=jnp.float32)
                    + c3b[...][None, :])


def _pool_mlp(h, batch, c1W, c1b, c2W, c2b, c3W, c3b):
    return pl.pallas_call(
        _pool_body,
        grid=(_NBLK,),
        in_specs=[
            pl.BlockSpec((_BLK, HID), lambda i: (i, 0)),
            pl.BlockSpec((_BLK, 1), lambda i: (i, 0)),
            pl.BlockSpec((2 * HID, HID), lambda i: (0, 0)),
            pl.BlockSpec((HID,), lambda i: (0,)),
            pl.BlockSpec((HID, HID // 2), lambda i: (0, 0)),
            pl.BlockSpec((HID // 2,), lambda i: (0,)),
            pl.BlockSpec((HID // 2, 10), lambda i: (0, 0)),
            pl.BlockSpec((10,), lambda i: (0,)),
        ],
        out_specs=pl.BlockSpec((G, 10), lambda i: (0, 0)),
        out_shape=jax.ShapeDtypeStruct((G, 10), jnp.float32),
        scratch_shapes=[
            pltpu.VMEM((G, HID), jnp.float32),
            pltpu.VMEM((G, HID), jnp.float32),
            pltpu.VMEM((G, HID), jnp.float32),
        ],
    )(h, batch, c1W, c1b, c2W, c2b, c3W, c3b)


def kernel(x, edge_index, batch,
           W0, as0, ad0, b0, g0, be0,
           W1, as1, ad1, b1, g1, be1,
           W2, as2, ad2, b2, g2, be2,
           W3, as3, ad3, b3, g3, be3,
           skip_W, skip_b, c1W, c1b, c2W, c2b, c3W, c3b):
    dst2 = edge_index[1].reshape(EROWS, K)
    packed = edge_index[0] | (edge_index[1] << 16)
    zeros = jnp.zeros((N, EXT), jnp.float32)

    Ws = [W0, W1, W2, W3]
    ass = [as0, as1, as2, as3]
    ads = [ad0, ad1, ad2, ad3]
    bs = [b0, b1, b2, b3]
    gs = [g0, g1, g2, g3]
    bes = [be0, be1, be2, be3]

    skip = _skip(x, skip_W, skip_b)
    h = x
    atts = []
    for i in range(4):
        hext, asrc, adst = _pre(h, Ws[i], ass[i], ads[i])
        msg, ex2 = _edge_pass(hext, packed, adst.reshape(N), zeros)
        res = skip if i == 0 else h
        hnext, inv, attl = _post(msg, hext, asrc, adst, res,
                                 bs[i], gs[i], bes[i], last=(i == 3))
        att_e = _att_pass(ex2, dst2, inv.reshape(N))
        atts.append(jnp.concatenate([att_e.reshape(E),
                                     attl.reshape(N)])[:, None])
        h = hnext

    out = _pool_mlp(h, batch.reshape(N, 1), c1W, c1b, c2W, c2b, c3W, c3b)
    return (out,) + tuple(atts)
